# Initial kernel scaffold; baseline (speedup 1.0000x reference)
import jax
import jax.numpy as jnp
from jax.experimental import pallas as pl


def kernel(x, edge_index, W1, b1, W2, b2, W3, b3):
    raise NotImplementedError("write your pallas kernel here")


# speedup vs baseline: 16.5254x; 16.5254x over previous
import functools

import jax
import jax.numpy as jnp
from jax import lax
from jax.experimental import pallas as pl
from jax.experimental.pallas import tpu as pltpu
from jax.experimental.pallas import tpu_sc as plsc

N = 10000
E = 320000
D = 128
NC = 2
NS = 16
NW = NC * NS
CH = 128
CPT = 80
EPT = CPT * CH
EP = NW * EPT
PAD = EP - E
RPT = 632
RS = NS * RPT
MB = 400
GRID = N // MB


def _sc_mesh():
    return plsc.VectorSubcoreMesh(core_axis_name="c", subcore_axis_name="s",
                                  num_cores=NC, num_subcores=NS)


@functools.cache
def _make_sc_deg():
    return pl.kernel(
        _sc_deg_body,
        out_type=jax.ShapeDtypeStruct((NC, RS, D), jnp.float32),
        mesh=_sc_mesh(),
        scratch_types=[
            pltpu.VMEM((CPT, CH), jnp.int32),
            pltpu.VMEM((CH, D), jnp.float32),
            pltpu.VMEM_SHARED((RS, D), jnp.float32),
        ],
    )


def _sc_deg_body(dstI, out, dstv, fill, shared):
    c = lax.axis_index("c")
    s = lax.axis_index("s")
    wid = c * NS + s

    pltpu.sync_copy(dstI.at[wid], dstv)

    def _fill_rows(r, val):
        for k in range(D // 16):
            fill[r, pl.ds(k * 16, 16)] = jnp.full((16,), val, jnp.float32)

    lax.fori_loop(0, CH, lambda r, _: (_fill_rows(r, 0.0), 0)[1], 0)
    base = s * RPT
    for i in range(4):
        pltpu.sync_copy(fill, shared.at[pl.ds(base + i * CH, CH)])
    pltpu.sync_copy(fill.at[pl.ds(0, 120)], shared.at[pl.ds(base + 512, 120)])

    lax.fori_loop(0, CH, lambda r, _: (_fill_rows(r, 1.0), 0)[1], 0)
    plsc.subcore_barrier()

    def body(j, _):
        pltpu.sync_copy(fill, shared.at[dstv.at[j]], add=True)
        return 0

    lax.fori_loop(0, CPT, body, 0)
    plsc.subcore_barrier()

    pltpu.sync_copy(shared.at[pl.ds(base, RPT)], out.at[c, pl.ds(base, RPT)])


@functools.cache
def _make_sc_agg():
    return pl.kernel(
        _sc_agg_body,
        out_type=jax.ShapeDtypeStruct((NC, RS, D), jnp.float32),
        mesh=_sc_mesh(),
        scratch_types=[
            pltpu.VMEM((CPT, CH), jnp.int32),
            pltpu.VMEM((CPT, CH), jnp.int32),
            pltpu.VMEM((CH, D), jnp.float32),
            pltpu.VMEM_SHARED((RS, D), jnp.float32),
            pltpu.SemaphoreType.DMA,
        ],
    )


def _sc_agg_body(h, srcI, dstI, out, srcv, dstv, rowsA, shared, semA):
    c = lax.axis_index("c")
    s = lax.axis_index("s")
    wid = c * NS + s

    pltpu.sync_copy(srcI.at[wid], srcv)
    pltpu.sync_copy(dstI.at[wid], dstv)

    def _zero_row(r, _):
        for k in range(D // 16):
            rowsA[r, pl.ds(k * 16, 16)] = jnp.zeros((16,), jnp.float32)
        return 0

    lax.fori_loop(0, CH, _zero_row, 0)
    base = s * RPT
    for i in range(4):
        pltpu.sync_copy(rowsA, shared.at[pl.ds(base + i * CH, CH)])
    pltpu.sync_copy(rowsA.at[pl.ds(0, 120)], shared.at[pl.ds(base + 512, 120)])
    plsc.subcore_barrier()

    def body(j, _):
        pltpu.async_copy(h.at[srcv.at[j]], rowsA, semA).wait()
        pltpu.sync_copy(rowsA, shared.at[dstv.at[j]], add=True)
        return 0

    lax.fori_loop(0, CPT, body, 0)
    plsc.subcore_barrier()

    pltpu.sync_copy(shared.at[pl.ds(base, RPT)], out.at[c, pl.ds(base, RPT)])


def _dinv_of(d0, d1):
    return lax.rsqrt(1.0 + d0[:, :1] + d1[:, :1])


def _t1_body(d0, d1, x, W, o):
    dinv = _dinv_of(d0, d1)
    o[:] = jnp.dot(x[:] * dinv, W[:], preferred_element_type=jnp.float32)


def _t2_body(d0, d1, p0, p1, h, b, W, o):
    dinv = _dinv_of(d0, d1)
    agg = p0[:] + p1[:] + h[:]
    xn = jnp.maximum(agg * dinv + b[:], 0.0)
    o[:] = jnp.dot(xn * dinv, W[:], preferred_element_type=jnp.float32)


def _t3_body(d0, d1, p0, p1, h, b, o):
    dinv = _dinv_of(d0, d1)
    o[:] = (p0[:] + p1[:] + h[:]) * dinv + b[:]


_deg_spec = pl.BlockSpec((MB, D), lambda i: (i, 0))
_row_spec = pl.BlockSpec((MB, D), lambda i: (i, 0))
_w_spec = pl.BlockSpec((D, D), lambda i: (0, 0))
_b_spec = pl.BlockSpec((1, D), lambda i: (0, 0))
_out_f32 = jax.ShapeDtypeStruct((N, D), jnp.float32)

_t1 = pl.pallas_call(
    _t1_body, grid=(GRID,),
    in_specs=[_deg_spec, _deg_spec, _row_spec, _w_spec],
    out_specs=_row_spec, out_shape=_out_f32)

_t2 = pl.pallas_call(
    _t2_body, grid=(GRID,),
    in_specs=[_deg_spec, _deg_spec, _row_spec, _row_spec, _row_spec,
              _b_spec, _w_spec],
    out_specs=_row_spec, out_shape=_out_f32)

_t3 = pl.pallas_call(
    _t3_body, grid=(GRID,),
    in_specs=[_deg_spec, _deg_spec, _row_spec, _row_spec, _row_spec, _b_spec],
    out_specs=_row_spec, out_shape=_out_f32)


@jax.jit
def kernel(x, edge_index, W1, b1, W2, b2, W3, b3):
    src = edge_index[0].astype(jnp.int32)
    dst = edge_index[1].astype(jnp.int32)
    i = jnp.arange(PAD, dtype=jnp.int32)
    srcP = jnp.concatenate([src, (i * 37) % N]).reshape(NW, CPT, CH)
    dstP = jnp.concatenate([dst, N + i % (RS - N)]).reshape(NW, CPT, CH)

    _sc_deg, _sc_agg = _make_sc_deg(), _make_sc_agg()
    degp = _sc_deg(dstP)
    d0, d1 = degp[0, :N], degp[1, :N]

    h1 = _t1(d0, d1, x, W1)
    p = _sc_agg(h1, srcP, dstP)
    h2 = _t2(d0, d1, p[0, :N], p[1, :N], h1, b1.reshape(1, D), W2)
    p = _sc_agg(h2, srcP, dstP)
    h3 = _t2(d0, d1, p[0, :N], p[1, :N], h2, b2.reshape(1, D), W3)
    p = _sc_agg(h3, srcP, dstP)
    return _t3(d0, d1, p[0, :N], p[1, :N], h3, b3.reshape(1, D))


# speedup vs baseline: 18.5125x; 1.1202x over previous
import functools

import jax
import jax.numpy as jnp
from jax import lax
from jax.experimental import pallas as pl
from jax.experimental.pallas import tpu as pltpu
from jax.experimental.pallas import tpu_sc as plsc

N = 10000
E = 320000
D = 128
NC = 2
NS = 16
NW = NC * NS
CH = 128
CPT = 80
EPT = CPT * CH
EP = NW * EPT
PAD = EP - E
RPT = 632
RS = NS * RPT
MB = 400
GRID = N // MB


def _sc_mesh():
    return plsc.VectorSubcoreMesh(core_axis_name="c", subcore_axis_name="s",
                                  num_cores=NC, num_subcores=NS)


_PACK = 16384


@functools.cache
def _make_sc_deg():
    return pl.kernel(
        _sc_deg_body,
        out_type=jax.ShapeDtypeStruct((NC, RS, D), jnp.float32),
        mesh=_sc_mesh(),
        scratch_types=[
            pltpu.VMEM((CPT, CH), jnp.int32),
            pltpu.VMEM((1, CH), jnp.int32),
            pltpu.VMEM((CH, D), jnp.float32),
            pltpu.VMEM_SHARED((RS, D), jnp.float32),
        ],
    )


def _sc_deg_body(pidxI, out, pidx, didx, fill, shared):
    c = lax.axis_index("c")
    s = lax.axis_index("s")
    wid = c * NS + s

    pltpu.sync_copy(pidxI.at[wid], pidx)

    def _fill_rows(r, val):
        for k in range(D // 16):
            fill[r, pl.ds(k * 16, 16)] = jnp.full((16,), val, jnp.float32)

    lax.fori_loop(0, CH, lambda r, _: (_fill_rows(r, 0.0), 0)[1], 0)
    base = s * RPT
    for i in range(4):
        pltpu.sync_copy(fill, shared.at[pl.ds(base + i * CH, CH)])
    pltpu.sync_copy(fill.at[pl.ds(0, 120)], shared.at[pl.ds(base + 512, 120)])

    lax.fori_loop(0, CH, lambda r, _: (_fill_rows(r, 1.0), 0)[1], 0)
    plsc.subcore_barrier()

    def body(j, _):
        for k in range(CH // 16):
            v = pidx[j, pl.ds(k * 16, 16)]
            didx[0, pl.ds(k * 16, 16)] = lax.shift_right_logical(v, 14)
        pltpu.sync_copy(fill, shared.at[didx.at[0]], add=True)
        return 0

    lax.fori_loop(0, CPT, body, 0)
    plsc.subcore_barrier()

    pltpu.sync_copy(shared.at[pl.ds(base, RPT)], out.at[c, pl.ds(base, RPT)])


@functools.cache
def _make_sc_agg():
    return pl.kernel(
        _sc_agg_body,
        out_type=jax.ShapeDtypeStruct((NC, RS, D), jnp.float32),
        mesh=_sc_mesh(),
        scratch_types=[
            pltpu.VMEM((CPT, CH), jnp.int32),
            pltpu.VMEM((2, CH), jnp.int32),
            pltpu.VMEM((1, CH), jnp.int32),
            pltpu.VMEM((CH, D), jnp.float32),
            pltpu.VMEM((CH, D), jnp.float32),
            pltpu.VMEM_SHARED((RS, D), jnp.float32),
            pltpu.SemaphoreType.DMA,
            pltpu.SemaphoreType.DMA,
        ],
    )


def _sc_agg_body(h, pidxI, out, pidx, sidx, didx, rowsA, rowsB, shared,
                 semA, semB):
    c = lax.axis_index("c")
    s = lax.axis_index("s")
    wid = c * NS + s

    pltpu.sync_copy(pidxI.at[wid], pidx)

    def _zero_row(r, _):
        for k in range(D // 16):
            rowsA[r, pl.ds(k * 16, 16)] = jnp.zeros((16,), jnp.float32)
        return 0

    lax.fori_loop(0, CH, _zero_row, 0)
    base = s * RPT
    for i in range(4):
        pltpu.sync_copy(rowsA, shared.at[pl.ds(base + i * CH, CH)])
    pltpu.sync_copy(rowsA.at[pl.ds(0, 120)], shared.at[pl.ds(base + 512, 120)])
    plsc.subcore_barrier()

    def unpack_src(j, buf):
        for k in range(CH // 16):
            v = pidx[j, pl.ds(k * 16, 16)]
            sidx[buf, pl.ds(k * 16, 16)] = jnp.bitwise_and(v, _PACK - 1)

    def unpack_dst(j):
        for k in range(CH // 16):
            v = pidx[j, pl.ds(k * 16, 16)]
            didx[0, pl.ds(k * 16, 16)] = lax.shift_right_logical(v, 14)

    unpack_src(0, 0)
    pltpu.async_copy(h.at[sidx.at[0]], rowsA, semA)

    def body(j, _):
        def step(cur, sem_cur, nxt, sem_nxt, cbuf, nbuf):
            pltpu.make_async_copy(h.at[sidx.at[cbuf]], cur, sem_cur).wait()

            @pl.when(j + 1 < CPT)
            def _():
                unpack_src(j + 1, nbuf)
                pltpu.async_copy(h.at[sidx.at[nbuf]], nxt, sem_nxt)

            unpack_dst(j)
            pltpu.sync_copy(cur, shared.at[didx.at[0]], add=True)

        @pl.when(j % 2 == 0)
        def _():
            step(rowsA, semA, rowsB, semB, 0, 1)

        @pl.when(j % 2 == 1)
        def _():
            step(rowsB, semB, rowsA, semA, 1, 0)

        return 0

    lax.fori_loop(0, CPT, body, 0)
    plsc.subcore_barrier()

    pltpu.sync_copy(shared.at[pl.ds(base, RPT)], out.at[c, pl.ds(base, RPT)])


def _dinv_of(dd):
    return lax.rsqrt(1.0 + dd[:, 0:1] + dd[:, 1:2])


def _t1_body(dd, x, W, o):
    dinv = _dinv_of(dd)
    o[:] = jnp.dot(x[:] * dinv, W[:], preferred_element_type=jnp.float32)


def _t2_body(dd, p0, p1, h, b, W, o):
    dinv = _dinv_of(dd)
    agg = p0[:] + p1[:] + h[:]
    xn = jnp.maximum(agg * dinv + b[:], 0.0)
    o[:] = jnp.dot(xn * dinv, W[:], preferred_element_type=jnp.float32)


def _t3_body(dd, p0, p1, h, b, o):
    dinv = _dinv_of(dd)
    o[:] = (p0[:] + p1[:] + h[:]) * dinv + b[:]


_deg_spec = pl.BlockSpec((MB, NC), lambda i: (i, 0))
_row_spec = pl.BlockSpec((MB, D), lambda i: (i, 0))
_w_spec = pl.BlockSpec((D, D), lambda i: (0, 0))
_b_spec = pl.BlockSpec((1, D), lambda i: (0, 0))
_out_f32 = jax.ShapeDtypeStruct((N, D), jnp.float32)

_t1 = pl.pallas_call(
    _t1_body, grid=(GRID,),
    in_specs=[_deg_spec, _row_spec, _w_spec],
    out_specs=_row_spec, out_shape=_out_f32)

_t2 = pl.pallas_call(
    _t2_body, grid=(GRID,),
    in_specs=[_deg_spec, _row_spec, _row_spec, _row_spec,
              _b_spec, _w_spec],
    out_specs=_row_spec, out_shape=_out_f32)

_t3 = pl.pallas_call(
    _t3_body, grid=(GRID,),
    in_specs=[_deg_spec, _row_spec, _row_spec, _row_spec, _b_spec],
    out_specs=_row_spec, out_shape=_out_f32)


@jax.jit
def kernel(x, edge_index, W1, b1, W2, b2, W3, b3):
    src = edge_index[0].astype(jnp.int32)
    dst = edge_index[1].astype(jnp.int32)
    i = jnp.arange(PAD, dtype=jnp.int32)
    srcP = jnp.concatenate([src, (i * 37) % N])
    dstP = jnp.concatenate([dst, N + i % (RS - N)])
    pidxP = (srcP + dstP * _PACK).reshape(NW, CPT, CH)

    _sc_deg, _sc_agg = _make_sc_deg(), _make_sc_agg()
    degp = _sc_deg(pidxP)
    dd = degp[:, :N, 0].T

    h1 = _t1(dd, x, W1)
    p = _sc_agg(h1, pidxP)
    h2 = _t2(dd, p[0, :N], p[1, :N], h1, b1.reshape(1, D), W2)
    p = _sc_agg(h2, pidxP)
    h3 = _t2(dd, p[0, :N], p[1, :N], h2, b2.reshape(1, D), W3)
    p = _sc_agg(h3, pidxP)
    return _t3(dd, p[0, :N], p[1, :N], h3, b3.reshape(1, D))


# speedup vs baseline: 21.4421x; 1.1583x over previous
import functools

import jax
import jax.numpy as jnp
from jax import lax
from jax.experimental import pallas as pl
from jax.experimental.pallas import tpu as pltpu
from jax.experimental.pallas import tpu_sc as plsc

N = 10000
E = 320000
D = 128
NC = 2
NS = 16
NW = NC * NS
CH = 128
CPT = 80
EPT = CPT * CH
EP = NW * EPT
PAD = EP - E
RPT = 632
RS = NS * RPT
MB = 400
GRID = N // MB


def _sc_mesh():
    return plsc.VectorSubcoreMesh(core_axis_name="c", subcore_axis_name="s",
                                  num_cores=NC, num_subcores=NS)


_PACK = 16384


@functools.cache
def _make_sc_deg():
    return pl.kernel(
        _sc_deg_body,
        out_type=jax.ShapeDtypeStruct((NC, RS, D), jnp.float32),
        mesh=_sc_mesh(),
        scratch_types=[
            pltpu.VMEM((CPT, CH), jnp.int32),
            pltpu.VMEM((1, CH), jnp.int32),
            pltpu.VMEM((CH, D), jnp.float32),
            pltpu.VMEM_SHARED((RS, D), jnp.float32),
        ],
    )


def _sc_deg_body(pidxI, out, pidx, didx, fill, shared):
    c = lax.axis_index("c")
    s = lax.axis_index("s")
    wid = c * NS + s

    pltpu.sync_copy(pidxI.at[wid], pidx)

    def _fill_rows(r, val):
        for k in range(D // 16):
            fill[r, pl.ds(k * 16, 16)] = jnp.full((16,), val, jnp.float32)

    lax.fori_loop(0, CH, lambda r, _: (_fill_rows(r, 0.0), 0)[1], 0)
    base = s * RPT
    for i in range(4):
        pltpu.sync_copy(fill, shared.at[pl.ds(base + i * CH, CH)])
    pltpu.sync_copy(fill.at[pl.ds(0, 120)], shared.at[pl.ds(base + 512, 120)])

    lax.fori_loop(0, CH, lambda r, _: (_fill_rows(r, 1.0), 0)[1], 0)
    plsc.subcore_barrier()

    def body(j, _):
        for k in range(CH // 16):
            v = pidx[j, pl.ds(k * 16, 16)]
            didx[0, pl.ds(k * 16, 16)] = lax.shift_right_logical(v, 14)
        pltpu.sync_copy(fill, shared.at[didx.at[0]], add=True)
        return 0

    lax.fori_loop(0, CPT, body, 0)
    plsc.subcore_barrier()

    pltpu.sync_copy(shared.at[pl.ds(base, RPT)], out.at[c, pl.ds(base, RPT)])


@functools.cache
def _make_sc_agg():
    return pl.kernel(
        _sc_agg_body,
        out_type=jax.ShapeDtypeStruct((NC, RS, D), jnp.float32),
        mesh=_sc_mesh(),
        scratch_types=[
            pltpu.VMEM((CPT, CH), jnp.int32),
            pltpu.VMEM((2, CH), jnp.int32),
            pltpu.VMEM((1, CH), jnp.int32),
            pltpu.VMEM((CH, D), jnp.float32),
            pltpu.VMEM((CH, D), jnp.float32),
            pltpu.VMEM_SHARED((RS, D), jnp.float32),
            pltpu.SemaphoreType.DMA,
            pltpu.SemaphoreType.DMA,
        ],
    )


def _sc_agg_body(h, pidxI, out, pidx, sidx, didx, rowsA, rowsB, shared,
                 semA, semB):
    c = lax.axis_index("c")
    s = lax.axis_index("s")
    wid = c * NS + s

    pltpu.sync_copy(pidxI.at[wid], pidx)

    def _zero_row(r, _):
        for k in range(D // 16):
            rowsA[r, pl.ds(k * 16, 16)] = jnp.zeros((16,), jnp.float32)
        return 0

    lax.fori_loop(0, CH, _zero_row, 0)
    base = s * RPT
    for i in range(4):
        pltpu.sync_copy(rowsA, shared.at[pl.ds(base + i * CH, CH)])
    pltpu.sync_copy(rowsA.at[pl.ds(0, 120)], shared.at[pl.ds(base + 512, 120)])
    plsc.subcore_barrier()

    def unpack_src(j, buf):
        for k in range(CH // 16):
            v = pidx[j, pl.ds(k * 16, 16)]
            sidx[buf, pl.ds(k * 16, 16)] = jnp.bitwise_and(v, _PACK - 1)

    def unpack_dst(j):
        for k in range(CH // 16):
            v = pidx[j, pl.ds(k * 16, 16)]
            didx[0, pl.ds(k * 16, 16)] = lax.shift_right_logical(v, 14)

    unpack_src(0, 0)
    pltpu.async_copy(h.at[sidx.at[0]], rowsA, semA)

    def body(j, _):
        def step(cur, sem_cur, nxt, sem_nxt, cbuf, nbuf):
            pltpu.make_async_copy(h.at[sidx.at[cbuf]], cur, sem_cur).wait()

            @pl.when(j + 1 < CPT)
            def _():
                unpack_src(j + 1, nbuf)
                pltpu.async_copy(h.at[sidx.at[nbuf]], nxt, sem_nxt)

            unpack_dst(j)
            pltpu.sync_copy(cur, shared.at[didx.at[0]], add=True)

        @pl.when(j % 2 == 0)
        def _():
            step(rowsA, semA, rowsB, semB, 0, 1)

        @pl.when(j % 2 == 1)
        def _():
            step(rowsB, semB, rowsA, semA, 1, 0)

        return 0

    lax.fori_loop(0, CPT, body, 0)
    plsc.subcore_barrier()

    pltpu.sync_copy(shared.at[pl.ds(base, RPT)], out.at[c, pl.ds(base, RPT)])


def _t1_body(d0, d1, x, W, o, dv):
    dinv = lax.rsqrt(1.0 + d0[0][:, 0:1] + d1[0][:, 0:1])
    dv[:] = jnp.broadcast_to(dinv, (MB, D))
    o[:] = jnp.dot(x[:] * dinv, W[:], preferred_element_type=jnp.float32)


def _t2_body(dv, p0, p1, h, b, W, o):
    dinv = dv[:]
    agg = p0[0] + p1[0] + h[:]
    xn = jnp.maximum(agg * dinv + b[:], 0.0)
    o[:] = jnp.dot(xn * dinv, W[:], preferred_element_type=jnp.float32)


def _t3_body(dv, p0, p1, h, b, o):
    o[:] = (p0[0] + p1[0] + h[:]) * dv[:] + b[:]


_p0_spec = pl.BlockSpec((1, MB, D), lambda i: (0, i, 0))
_p1_spec = pl.BlockSpec((1, MB, D), lambda i: (1, i, 0))
_row_spec = pl.BlockSpec((MB, D), lambda i: (i, 0))
_w_spec = pl.BlockSpec((D, D), lambda i: (0, 0))
_b_spec = pl.BlockSpec((1, D), lambda i: (0, 0))
_out_f32 = jax.ShapeDtypeStruct((N, D), jnp.float32)

_t1 = pl.pallas_call(
    _t1_body, grid=(GRID,),
    in_specs=[_p0_spec, _p1_spec, _row_spec, _w_spec],
    out_specs=[_row_spec, _row_spec], out_shape=[_out_f32, _out_f32])

_t2 = pl.pallas_call(
    _t2_body, grid=(GRID,),
    in_specs=[_row_spec, _p0_spec, _p1_spec, _row_spec,
              _b_spec, _w_spec],
    out_specs=_row_spec, out_shape=_out_f32)

_t3 = pl.pallas_call(
    _t3_body, grid=(GRID,),
    in_specs=[_row_spec, _p0_spec, _p1_spec, _row_spec, _b_spec],
    out_specs=_row_spec, out_shape=_out_f32)


@jax.jit
def kernel(x, edge_index, W1, b1, W2, b2, W3, b3):
    src = edge_index[0].astype(jnp.int32)
    dst = edge_index[1].astype(jnp.int32)
    i = jnp.arange(PAD, dtype=jnp.int32)
    srcP = jnp.concatenate([src, (i * 37) % N])
    dstP = jnp.concatenate([dst, N + i % (RS - N)])
    pidxP = (srcP + dstP * _PACK).reshape(NW, CPT, CH)

    _sc_deg, _sc_agg = _make_sc_deg(), _make_sc_agg()
    degp = _sc_deg(pidxP)

    h1, dv = _t1(degp, degp, x, W1)
    p = _sc_agg(h1, pidxP)
    h2 = _t2(dv, p, p, h1, b1.reshape(1, D), W2)
    p = _sc_agg(h2, pidxP)
    h3 = _t2(dv, p, p, h2, b2.reshape(1, D), W3)
    p = _sc_agg(h3, pidxP)
    return _t3(dv, p, p, h3, b3.reshape(1, D))


# speedup vs baseline: 23.0925x; 1.0770x over previous
import functools

import jax
import jax.numpy as jnp
from jax import lax
from jax.experimental import pallas as pl
from jax.experimental.pallas import tpu as pltpu
from jax.experimental.pallas import tpu_sc as plsc

N = 10000
E = 320000
D = 128
NC = 2
NS = 16
NW = NC * NS
CH = 128
CPT = 80
EPT = CPT * CH
EP = NW * EPT
PAD = EP - E
RPT = 632
RS = NS * RPT
MB = 2000
GRID = N // MB


def _sc_mesh():
    return plsc.VectorSubcoreMesh(core_axis_name="c", subcore_axis_name="s",
                                  num_cores=NC, num_subcores=NS)


_PACK = 16384


@functools.cache
def _make_sc_deg():
    return pl.kernel(
        _sc_deg_body,
        out_type=jax.ShapeDtypeStruct((NC, RS, D), jnp.float32),
        mesh=_sc_mesh(),
        scratch_types=[
            pltpu.VMEM((CPT, CH), jnp.int32),
            pltpu.VMEM((1, CH), jnp.int32),
            pltpu.VMEM((CH, D), jnp.float32),
            pltpu.VMEM_SHARED((RS, D), jnp.float32),
        ],
    )


def _sc_deg_body(pidxI, out, pidx, didx, fill, shared):
    c = lax.axis_index("c")
    s = lax.axis_index("s")
    wid = c * NS + s

    pltpu.sync_copy(pidxI.at[wid], pidx)

    def _fill_rows(r, val):
        for k in range(D // 16):
            fill[r, pl.ds(k * 16, 16)] = jnp.full((16,), val, jnp.float32)

    lax.fori_loop(0, CH, lambda r, _: (_fill_rows(r, 0.0), 0)[1], 0)
    base = s * RPT
    for i in range(4):
        pltpu.sync_copy(fill, shared.at[pl.ds(base + i * CH, CH)])
    pltpu.sync_copy(fill.at[pl.ds(0, 120)], shared.at[pl.ds(base + 512, 120)])

    lax.fori_loop(0, CH, lambda r, _: (_fill_rows(r, 1.0), 0)[1], 0)
    plsc.subcore_barrier()

    def body(j, _):
        for k in range(CH // 16):
            v = pidx[j, pl.ds(k * 16, 16)]
            didx[0, pl.ds(k * 16, 16)] = lax.shift_right_logical(v, 14)
        pltpu.sync_copy(fill, shared.at[didx.at[0]], add=True)
        return 0

    lax.fori_loop(0, CPT, body, 0)
    plsc.subcore_barrier()

    pltpu.sync_copy(shared.at[pl.ds(base, RPT)], out.at[c, pl.ds(base, RPT)])


@functools.cache
def _make_sc_agg():
    return pl.kernel(
        _sc_agg_body,
        out_type=jax.ShapeDtypeStruct((NC, RS, D), jnp.float32),
        mesh=_sc_mesh(),
        scratch_types=[
            pltpu.VMEM((CPT, CH), jnp.int32),
            pltpu.VMEM((2, CH), jnp.int32),
            pltpu.VMEM((1, CH), jnp.int32),
            pltpu.VMEM((CH, D), jnp.float32),
            pltpu.VMEM((CH, D), jnp.float32),
            pltpu.VMEM_SHARED((RS, D), jnp.float32),
            pltpu.SemaphoreType.DMA,
            pltpu.SemaphoreType.DMA,
        ],
    )


def _sc_agg_body(h, pidxI, out, pidx, sidx, didx, rowsA, rowsB, shared,
                 semA, semB):
    c = lax.axis_index("c")
    s = lax.axis_index("s")
    wid = c * NS + s

    pltpu.sync_copy(pidxI.at[wid], pidx)

    def _zero_row(r, _):
        for k in range(D // 16):
            rowsA[r, pl.ds(k * 16, 16)] = jnp.zeros((16,), jnp.float32)
        return 0

    lax.fori_loop(0, CH, _zero_row, 0)
    base = s * RPT
    for i in range(4):
        pltpu.sync_copy(rowsA, shared.at[pl.ds(base + i * CH, CH)])
    pltpu.sync_copy(rowsA.at[pl.ds(0, 120)], shared.at[pl.ds(base + 512, 120)])
    plsc.subcore_barrier()

    def unpack_src(j, buf):
        for k in range(CH // 16):
            v = pidx[j, pl.ds(k * 16, 16)]
            sidx[buf, pl.ds(k * 16, 16)] = jnp.bitwise_and(v, _PACK - 1)

    def unpack_dst(j):
        for k in range(CH // 16):
            v = pidx[j, pl.ds(k * 16, 16)]
            didx[0, pl.ds(k * 16, 16)] = lax.shift_right_logical(v, 14)

    unpack_src(0, 0)
    pltpu.async_copy(h.at[sidx.at[0]], rowsA, semA)

    def body(j, _):
        def step(cur, sem_cur, nxt, sem_nxt, cbuf, nbuf):
            pltpu.make_async_copy(h.at[sidx.at[cbuf]], cur, sem_cur).wait()

            @pl.when(j + 1 < CPT)
            def _():
                unpack_src(j + 1, nbuf)
                pltpu.async_copy(h.at[sidx.at[nbuf]], nxt, sem_nxt)

            unpack_dst(j)
            pltpu.sync_copy(cur, shared.at[didx.at[0]], add=True)

        @pl.when(j % 2 == 0)
        def _():
            step(rowsA, semA, rowsB, semB, 0, 1)

        @pl.when(j % 2 == 1)
        def _():
            step(rowsB, semB, rowsA, semA, 1, 0)

        return 0

    lax.fori_loop(0, CPT, body, 0)
    plsc.subcore_barrier()

    pltpu.sync_copy(shared.at[pl.ds(base, RPT)], out.at[c, pl.ds(base, RPT)])


def _t1_body(d0, d1, x, W, o, dv):
    dinv = lax.rsqrt(1.0 + d0[0][:, 0:1] + d1[0][:, 0:1])
    dv[:] = jnp.broadcast_to(dinv, (MB, D))
    o[:] = jnp.dot(x[:] * dinv, W[:], preferred_element_type=jnp.float32)


def _t2_body(dv, p0, p1, h, b, W, o):
    dinv = dv[:]
    agg = p0[0] + p1[0] + h[:]
    xn = jnp.maximum(agg * dinv + b[:], 0.0)
    o[:] = jnp.dot(xn * dinv, W[:], preferred_element_type=jnp.float32)


def _t3_body(dv, p0, p1, h, b, o):
    o[:] = (p0[0] + p1[0] + h[:]) * dv[:] + b[:]


_p0_spec = pl.BlockSpec((1, MB, D), lambda i: (0, i, 0))
_p1_spec = pl.BlockSpec((1, MB, D), lambda i: (1, i, 0))
_row_spec = pl.BlockSpec((MB, D), lambda i: (i, 0))
_w_spec = pl.BlockSpec((D, D), lambda i: (0, 0))
_b_spec = pl.BlockSpec((1, D), lambda i: (0, 0))
_out_f32 = jax.ShapeDtypeStruct((N, D), jnp.float32)

_t1 = pl.pallas_call(
    _t1_body, grid=(GRID,),
    in_specs=[_p0_spec, _p1_spec, _row_spec, _w_spec],
    out_specs=[_row_spec, _row_spec], out_shape=[_out_f32, _out_f32])

_t2 = pl.pallas_call(
    _t2_body, grid=(GRID,),
    in_specs=[_row_spec, _p0_spec, _p1_spec, _row_spec,
              _b_spec, _w_spec],
    out_specs=_row_spec, out_shape=_out_f32)

_t3 = pl.pallas_call(
    _t3_body, grid=(GRID,),
    in_specs=[_row_spec, _p0_spec, _p1_spec, _row_spec, _b_spec],
    out_specs=_row_spec, out_shape=_out_f32)


@jax.jit
def kernel(x, edge_index, W1, b1, W2, b2, W3, b3):
    src = edge_index[0].astype(jnp.int32)
    dst = edge_index[1].astype(jnp.int32)
    i = jnp.arange(PAD, dtype=jnp.int32)
    srcP = jnp.concatenate([src, (i * 37) % N])
    dstP = jnp.concatenate([dst, N + i % (RS - N)])
    pidxP = (srcP + dstP * _PACK).reshape(NW, CPT, CH)

    _sc_deg, _sc_agg = _make_sc_deg(), _make_sc_agg()
    degp = _sc_deg(pidxP)

    h1, dv = _t1(degp, degp, x, W1)
    p = _sc_agg(h1, pidxP)
    h2 = _t2(dv, p, p, h1, b1.reshape(1, D), W2)
    p = _sc_agg(h2, pidxP)
    h3 = _t2(dv, p, p, h2, b2.reshape(1, D), W3)
    p = _sc_agg(h3, pidxP)
    return _t3(dv, p, p, h3, b3.reshape(1, D))


# speedup vs baseline: 23.2000x; 1.0047x over previous
import functools

import jax
import jax.numpy as jnp
from jax import lax
from jax.experimental import pallas as pl
from jax.experimental.pallas import tpu as pltpu
from jax.experimental.pallas import tpu_sc as plsc

N = 10000
E = 320000
D = 128
NC = 2
NS = 16
NW = NC * NS
CH = 128
CPT = 80
EPT = CPT * CH
EP = NW * EPT
PAD = EP - E
RPT = 632
RS = NS * RPT
MB = 2000
GRID = N // MB


def _sc_mesh():
    return plsc.VectorSubcoreMesh(core_axis_name="c", subcore_axis_name="s",
                                  num_cores=NC, num_subcores=NS)


_PACK = 16384


@functools.cache
def _make_sc_deg():
    return pl.kernel(
        _sc_deg_body,
        out_type=jax.ShapeDtypeStruct((NC, RS, D), jnp.float32),
        mesh=_sc_mesh(),
        scratch_types=[
            pltpu.VMEM((CPT, CH), jnp.int32),
            pltpu.VMEM((1, CH), jnp.int32),
            pltpu.VMEM((CH, D), jnp.float32),
            pltpu.VMEM_SHARED((RS, D), jnp.float32),
        ],
    )


def _sc_deg_body(pidxI, out, pidx, didx, fill, shared):
    c = lax.axis_index("c")
    s = lax.axis_index("s")
    wid = c * NS + s

    pltpu.sync_copy(pidxI.at[wid], pidx)

    def _fill_rows(r, val):
        for k in range(D // 16):
            fill[r, pl.ds(k * 16, 16)] = jnp.full((16,), val, jnp.float32)

    lax.fori_loop(0, CH, lambda r, _: (_fill_rows(r, 0.0), 0)[1], 0)
    base = s * RPT
    for i in range(4):
        pltpu.sync_copy(fill, shared.at[pl.ds(base + i * CH, CH)])
    pltpu.sync_copy(fill.at[pl.ds(0, 120)], shared.at[pl.ds(base + 512, 120)])

    lax.fori_loop(0, CH, lambda r, _: (_fill_rows(r, 1.0), 0)[1], 0)
    plsc.subcore_barrier()

    def body(j, _):
        for k in range(CH // 16):
            v = pidx[j, pl.ds(k * 16, 16)]
            didx[0, pl.ds(k * 16, 16)] = lax.shift_right_logical(v, 14)
        pltpu.sync_copy(fill, shared.at[didx.at[0]], add=True)
        return 0

    lax.fori_loop(0, CPT, body, 0)
    plsc.subcore_barrier()

    pltpu.sync_copy(shared.at[pl.ds(base, RPT)], out.at[c, pl.ds(base, RPT)])


@functools.cache
def _make_sc_agg():
    return pl.kernel(
        _sc_agg_body,
        out_type=jax.ShapeDtypeStruct((NC, RS, D), jnp.float32),
        mesh=_sc_mesh(),
        scratch_types=[
            pltpu.VMEM((CPT, CH), jnp.int32),
            pltpu.VMEM((2, CH), jnp.int32),
            pltpu.VMEM((1, CH), jnp.int32),
            pltpu.VMEM((CH, D), jnp.float32),
            pltpu.VMEM((CH, D), jnp.float32),
            pltpu.VMEM_SHARED((RS, D), jnp.float32),
            pltpu.SemaphoreType.DMA,
            pltpu.SemaphoreType.DMA,
        ],
    )


def _sc_agg_body(h, pidxI, out, pidx, sidx, didx, rowsA, rowsB, shared,
                 semA, semB):
    c = lax.axis_index("c")
    s = lax.axis_index("s")
    wid = c * NS + s

    pltpu.sync_copy(pidxI.at[wid], pidx)

    def _zero_row(r, _):
        for k in range(D // 16):
            rowsA[r, pl.ds(k * 16, 16)] = jnp.zeros((16,), jnp.float32)
        return 0

    lax.fori_loop(0, CH, _zero_row, 0)
    base = s * RPT
    for i in range(4):
        pltpu.sync_copy(rowsA, shared.at[pl.ds(base + i * CH, CH)])
    pltpu.sync_copy(rowsA.at[pl.ds(0, 120)], shared.at[pl.ds(base + 512, 120)])
    plsc.subcore_barrier()

    def unpack_src(j, buf):
        for k in range(CH // 16):
            v = pidx[j, pl.ds(k * 16, 16)]
            sidx[buf, pl.ds(k * 16, 16)] = jnp.bitwise_and(v, _PACK - 1)

    def unpack_dst(j):
        for k in range(CH // 16):
            v = pidx[j, pl.ds(k * 16, 16)]
            didx[0, pl.ds(k * 16, 16)] = lax.shift_right_logical(v, 14)

    unpack_src(0, 0)
    pltpu.async_copy(h.at[sidx.at[0]], rowsA, semA)

    def body(j, _):
        def step(cur, sem_cur, nxt, sem_nxt, cbuf, nbuf):
            pltpu.make_async_copy(h.at[sidx.at[cbuf]], cur, sem_cur).wait()

            @pl.when(j + 1 < CPT)
            def _():
                unpack_src(j + 1, nbuf)
                pltpu.async_copy(h.at[sidx.at[nbuf]], nxt, sem_nxt)

            unpack_dst(j)
            pltpu.sync_copy(cur, shared.at[didx.at[0]], add=True)

        @pl.when(j % 2 == 0)
        def _():
            step(rowsA, semA, rowsB, semB, 0, 1)

        @pl.when(j % 2 == 1)
        def _():
            step(rowsB, semB, rowsA, semA, 1, 0)

        return 0

    lax.fori_loop(0, CPT, body, 0)
    plsc.subcore_barrier()

    pltpu.sync_copy(shared.at[pl.ds(base, RPT)], out.at[c, pl.ds(base, RPT)])


def _t1_body(d0, d1, x, W, o, dv):
    dinv = lax.rsqrt(1.0 + d0[0][:, 0:1] + d1[0][:, 0:1])
    dv[:] = jnp.broadcast_to(dinv, (MB, D))
    o[:] = jnp.dot(x[:] * dinv, W[:], preferred_element_type=jnp.float32)


def _t2_body(dv, p0, p1, h, b, W, o):
    dinv = dv[:]
    agg = p0[0] + p1[0] + h[:]
    xn = jnp.maximum(agg * dinv + b[:], 0.0)
    o[:] = jnp.dot(xn * dinv, W[:], preferred_element_type=jnp.float32)


def _t3_body(dv, p0, p1, h, b, o):
    o[:] = (p0[0] + p1[0] + h[:]) * dv[:] + b[:]


_PK_R = 256
_PK_E = _PK_R * CH
_PKGRID = EP // _PK_E


def _pack_body(src, dst, o):
    i = pl.program_id(0)
    r = jax.lax.broadcasted_iota(jnp.int32, (_PK_R, CH), 0)
    cix = jax.lax.broadcasted_iota(jnp.int32, (_PK_R, CH), 1)
    e = i * _PK_E + r * CH + cix
    pad = e >= E
    j = e - E
    s = jnp.where(pad, (j * 37) % N, src[0])
    d = jnp.where(pad, N + j % (RS - N), dst[0])
    o[:] = s + d * _PACK


_pack = pl.pallas_call(
    _pack_body, grid=(_PKGRID,),
    in_specs=[pl.BlockSpec((1, _PK_R, CH), lambda i: (0, i, 0)),
              pl.BlockSpec((1, _PK_R, CH), lambda i: (1, i, 0))],
    out_specs=pl.BlockSpec((_PK_R, CH), lambda i: (i, 0)),
    out_shape=jax.ShapeDtypeStruct((EP // CH, CH), jnp.int32))


_p0_spec = pl.BlockSpec((1, MB, D), lambda i: (0, i, 0))
_p1_spec = pl.BlockSpec((1, MB, D), lambda i: (1, i, 0))
_row_spec = pl.BlockSpec((MB, D), lambda i: (i, 0))
_w_spec = pl.BlockSpec((D, D), lambda i: (0, 0))
_b_spec = pl.BlockSpec((1, D), lambda i: (0, 0))
_out_f32 = jax.ShapeDtypeStruct((N, D), jnp.float32)

_t1 = pl.pallas_call(
    _t1_body, grid=(GRID,),
    in_specs=[_p0_spec, _p1_spec, _row_spec, _w_spec],
    out_specs=[_row_spec, _row_spec], out_shape=[_out_f32, _out_f32])

_t2 = pl.pallas_call(
    _t2_body, grid=(GRID,),
    in_specs=[_row_spec, _p0_spec, _p1_spec, _row_spec,
              _b_spec, _w_spec],
    out_specs=_row_spec, out_shape=_out_f32)

_t3 = pl.pallas_call(
    _t3_body, grid=(GRID,),
    in_specs=[_row_spec, _p0_spec, _p1_spec, _row_spec, _b_spec],
    out_specs=_row_spec, out_shape=_out_f32)


@jax.jit
def kernel(x, edge_index, W1, b1, W2, b2, W3, b3):
    ei3 = edge_index.astype(jnp.int32).reshape(2, E // CH, CH)
    pidxP = _pack(ei3, ei3).reshape(NW, CPT, CH)

    _sc_deg, _sc_agg = _make_sc_deg(), _make_sc_agg()
    degp = _sc_deg(pidxP)

    h1, dv = _t1(degp, degp, x, W1)
    p = _sc_agg(h1, pidxP)
    h2 = _t2(dv, p, p, h1, b1.reshape(1, D), W2)
    p = _sc_agg(h2, pidxP)
    h3 = _t2(dv, p, p, h2, b2.reshape(1, D), W3)
    p = _sc_agg(h3, pidxP)
    return _t3(dv, p, p, h3, b3.reshape(1, D))
